# baseline (speedup 1.0000x reference)
import functools

import jax
import jax.numpy as jnp
from jax import lax
from jax.experimental import pallas as pl
from jax.experimental.pallas import tpu as pltpu
from jax.experimental.pallas import tpu_sc as plsc

_B, _T, _D = 4096, 200, 64
_ROW = _T * _D
_NW = 32
_RPW = _B // _NW
_L = 16


def _compute_row(r, be_v, tok_v, obuf):
    brow = [be_v[pl.ds(r * _D + j * _L, _L)] for j in range(_D // _L)]

    def body(t, c):
        off = t * _D
        for j in range(_D // _L):
            s = off + j * _L
            obuf[pl.ds(s, _L)] = tok_v[pl.ds(s, _L)] + brow[j]
        return c

    lax.fori_loop(0, _T, body, 0, unroll=4)


def _sc_body(be_hbm, te_hbm, out_hbm, tok_v, be_v, obuf0, obuf1, sem0, sem1):
    nc = 2
    wid = lax.axis_index("s") * nc + lax.axis_index("c")
    base = wid * _RPW
    pltpu.sync_copy(te_hbm, tok_v)
    pltpu.sync_copy(be_hbm.at[pl.ds(base * _D, _RPW * _D)], be_v)

    def pair(i, c):
        r0 = 2 * i
        r1 = r0 + 1

        @pl.when(i > 0)
        def _():
            pltpu.make_async_copy(
                obuf0, out_hbm.at[pl.ds(base * _ROW, _ROW)], sem0).wait()

        _compute_row(r0, be_v, tok_v, obuf0)
        pltpu.async_copy(obuf0, out_hbm.at[pl.ds((base + r0) * _ROW, _ROW)], sem0)

        @pl.when(i > 0)
        def _():
            pltpu.make_async_copy(
                obuf1, out_hbm.at[pl.ds(base * _ROW, _ROW)], sem1).wait()

        _compute_row(r1, be_v, tok_v, obuf1)
        pltpu.async_copy(obuf1, out_hbm.at[pl.ds((base + r1) * _ROW, _ROW)], sem1)
        return c

    lax.fori_loop(0, _RPW // 2, pair, 0)
    pltpu.make_async_copy(obuf0, out_hbm.at[pl.ds(0, _ROW)], sem0).wait()
    pltpu.make_async_copy(obuf1, out_hbm.at[pl.ds(0, _ROW)], sem1).wait()


_sc_kernel = functools.partial(
    pl.kernel,
    out_type=jax.ShapeDtypeStruct((_B * _T * _D,), jnp.float32),
    mesh=plsc.VectorSubcoreMesh(core_axis_name="c", subcore_axis_name="s"),
    scratch_types=[
        pltpu.VMEM((_ROW,), jnp.float32),
        pltpu.VMEM((_RPW * _D,), jnp.float32),
        pltpu.VMEM((_ROW,), jnp.float32),
        pltpu.VMEM((_ROW,), jnp.float32),
        pltpu.SemaphoreType.DMA,
        pltpu.SemaphoreType.DMA,
    ],
)(_sc_body)


def kernel(x, batch_embed, token_embed):
    del x
    out_flat = _sc_kernel(batch_embed.reshape(-1), token_embed.reshape(-1))
    return out_flat.reshape(_B, _T, _D)


# speedup vs baseline: 1.7095x; 1.7095x over previous
import functools

import jax
import jax.numpy as jnp
from jax import lax
from jax.experimental import pallas as pl
from jax.experimental.pallas import tpu as pltpu
from jax.experimental.pallas import tpu_sc as plsc

_B, _T, _D = 4096, 200, 64
_ROW = _T * _D
_NW = 32
_RPW = _B // _NW
_L = 16


def _compute_row(r, be_v, tok_v, obuf):
    brow = [be_v[pl.ds(r * _D + j * _L, _L)] for j in range(_D // _L)]

    @plsc.parallel_loop(0, _T, unroll=4)
    def _(t):
        off = t * _D
        for j in range(_D // _L):
            s = off + j * _L
            obuf[pl.ds(s, _L)] = tok_v[pl.ds(s, _L)] + brow[j]


def _sc_body(be_hbm, te_hbm, out_hbm, tok_v, be_v, obuf0, obuf1, sem0, sem1):
    nc = 2
    wid = lax.axis_index("s") * nc + lax.axis_index("c")
    base = wid * _RPW
    pltpu.sync_copy(te_hbm, tok_v)
    pltpu.sync_copy(be_hbm.at[pl.ds(base * _D, _RPW * _D)], be_v)

    def pair(i, c):
        r0 = 2 * i
        r1 = r0 + 1

        @pl.when(i > 0)
        def _():
            pltpu.make_async_copy(
                obuf0, out_hbm.at[pl.ds(base * _ROW, _ROW)], sem0).wait()

        _compute_row(r0, be_v, tok_v, obuf0)
        pltpu.async_copy(obuf0, out_hbm.at[pl.ds((base + r0) * _ROW, _ROW)], sem0)

        @pl.when(i > 0)
        def _():
            pltpu.make_async_copy(
                obuf1, out_hbm.at[pl.ds(base * _ROW, _ROW)], sem1).wait()

        _compute_row(r1, be_v, tok_v, obuf1)
        pltpu.async_copy(obuf1, out_hbm.at[pl.ds((base + r1) * _ROW, _ROW)], sem1)
        return c

    lax.fori_loop(0, _RPW // 2, pair, 0)
    pltpu.make_async_copy(obuf0, out_hbm.at[pl.ds(0, _ROW)], sem0).wait()
    pltpu.make_async_copy(obuf1, out_hbm.at[pl.ds(0, _ROW)], sem1).wait()


_sc_kernel = functools.partial(
    pl.kernel,
    out_type=jax.ShapeDtypeStruct((_B * _T * _D,), jnp.float32),
    mesh=plsc.VectorSubcoreMesh(core_axis_name="c", subcore_axis_name="s"),
    scratch_types=[
        pltpu.VMEM((_ROW,), jnp.float32),
        pltpu.VMEM((_RPW * _D,), jnp.float32),
        pltpu.VMEM((_ROW,), jnp.float32),
        pltpu.VMEM((_ROW,), jnp.float32),
        pltpu.SemaphoreType.DMA,
        pltpu.SemaphoreType.DMA,
    ],
)(_sc_body)


def kernel(x, batch_embed, token_embed):
    del x
    out_flat = _sc_kernel(batch_embed.reshape(-1), token_embed.reshape(-1))
    return out_flat.reshape(_B, _T, _D)


# speedup vs baseline: 11.1783x; 6.5391x over previous
import functools

import jax
import jax.numpy as jnp
from jax import lax
from jax.experimental import pallas as pl
from jax.experimental.pallas import tpu as pltpu
from jax.experimental.pallas import tpu_sc as plsc

_B, _T, _D = 4096, 200, 64
_L = 16
_NDB = _D // 8
_NQ = 32 // _NDB
_TPQ = _T // _NQ
_G = _B // _L


def _compute_slab(t, tep_v, bt_v, obuf):
    tvec = tep_v[t, :]
    tv = [jnp.full((_L,), tvec[din]) for din in range(8)]

    @plsc.parallel_loop(0, _G, unroll=4)
    def _(g):
        s = g * _L
        for din in range(8):
            obuf[din, pl.ds(s, _L)] = bt_v[din, pl.ds(s, _L)] + tv[din]


def _sc_body(bt_hbm, tep_hbm, out_hbm, bt_v, tep_v, obuf0, obuf1, sem0, sem1):
    nc = 2
    wid = lax.axis_index("s") * nc + lax.axis_index("c")
    dblk = wid % _NDB
    t0 = (wid // _NDB) * _TPQ
    pltpu.sync_copy(bt_hbm.at[pl.ds(dblk * 8, 8), :], bt_v)
    pltpu.sync_copy(tep_hbm.at[dblk], tep_v)

    def pair(i, c):
        ta = t0 + 2 * i
        tb = ta + 1

        @pl.when(i > 0)
        def _():
            pltpu.make_async_copy(
                obuf0, out_hbm.at[t0, pl.ds(dblk * 8, 8), :], sem0).wait()

        _compute_slab(ta, tep_v, bt_v, obuf0)
        pltpu.async_copy(obuf0, out_hbm.at[ta, pl.ds(dblk * 8, 8), :], sem0)

        @pl.when(i > 0)
        def _():
            pltpu.make_async_copy(
                obuf1, out_hbm.at[t0, pl.ds(dblk * 8, 8), :], sem1).wait()

        _compute_slab(tb, tep_v, bt_v, obuf1)
        pltpu.async_copy(obuf1, out_hbm.at[tb, pl.ds(dblk * 8, 8), :], sem1)
        return c

    lax.fori_loop(0, _TPQ // 2, pair, 0)
    pltpu.make_async_copy(obuf0, out_hbm.at[0, pl.ds(0, 8), :], sem0).wait()
    pltpu.make_async_copy(obuf1, out_hbm.at[0, pl.ds(0, 8), :], sem1).wait()


_sc_kernel = functools.partial(
    pl.kernel,
    out_type=jax.ShapeDtypeStruct((_T, _D, _B), jnp.float32),
    mesh=plsc.VectorSubcoreMesh(core_axis_name="c", subcore_axis_name="s"),
    scratch_types=[
        pltpu.VMEM((8, _B), jnp.float32),
        pltpu.VMEM((_T, _L), jnp.float32),
        pltpu.VMEM((8, _B), jnp.float32),
        pltpu.VMEM((8, _B), jnp.float32),
        pltpu.SemaphoreType.DMA,
        pltpu.SemaphoreType.DMA,
    ],
    compiler_params=pltpu.CompilerParams(use_tc_tiling_on_sc=True),
)(_sc_body)


def kernel(x, batch_embed, token_embed):
    del x
    bt = batch_embed.T
    tep = token_embed.reshape(_T, _NDB, 8).transpose(1, 0, 2)
    tep = jnp.pad(tep, ((0, 0), (0, 0), (0, _L - 8)))
    z = _sc_kernel(bt, tep)
    return jnp.transpose(z, (2, 0, 1))
